# baseline (speedup 1.0000x reference)
import functools

import jax
import jax.numpy as jnp
from jax import lax
from jax.experimental import pallas as pl
from jax.experimental.pallas import tpu as pltpu
from jax.experimental.pallas import tpu_sc as plsc

NC, NS, L = 2, 16, 16
N = 10000
NP = 10240
E = 320000
E2 = 327680
D = 128
HID = 128
ROWS_PER_TILE = NP // NS
RB = 128
SEG = NP // NS
ED_DEG = E2 // (NC * NS)
CH_DEG = 2048
ED_PROP = E2 // NS
CK = 128
NFULL = ED_PROP // CK
PAIRS = NFULL // 2


def _mesh():
    return plsc.VectorSubcoreMesh(core_axis_name="c", subcore_axis_name="s",
                                  num_cores=NC, num_subcores=NS)


def _deg_body(src_hbm, ew_hbm, degp_hbm, acc, idx_buf, w_buf, shared, red_buf,
              sum_buf):
    c = lax.axis_index("c")
    s = lax.axis_index("s")
    wid = c * NS + s
    zero16 = jnp.zeros((L,), jnp.float32)

    def zacc(i, _):
        acc[pl.ds(i * L, L)] = zero16
        return 0
    lax.fori_loop(0, NP // L, zacc, 0, unroll=4)

    base = wid * ED_DEG

    def chunk(k, _):
        off = base + k * CH_DEG
        pltpu.sync_copy(src_hbm.at[pl.ds(off, CH_DEG)], idx_buf)
        pltpu.sync_copy(ew_hbm.at[pl.ds(off, CH_DEG)], w_buf)

        def inner(j, _):
            sl = pl.ds(j * L, L)
            plsc.addupdate_scatter(acc, [idx_buf[sl]], w_buf[sl])
            return 0
        lax.fori_loop(0, CH_DEG // L, inner, 0, unroll=4)
        return 0
    lax.fori_loop(0, ED_DEG // CH_DEG, chunk, 0)

    pltpu.sync_copy(acc, shared.at[s])
    plsc.subcore_barrier()

    def zsum(i, _):
        sum_buf[pl.ds(i * L, L)] = zero16
        return 0
    lax.fori_loop(0, SEG // L, zsum, 0, unroll=4)

    def redp(p, _):
        pltpu.sync_copy(shared.at[p, pl.ds(s * SEG, SEG)], red_buf)

        def addv(i, _):
            sl = pl.ds(i * L, L)
            sum_buf[sl] = sum_buf[sl] + red_buf[sl]
            return 0
        lax.fori_loop(0, SEG // L, addv, 0, unroll=4)
        return 0
    lax.fori_loop(0, NS, redp, 0)

    pltpu.sync_copy(sum_buf, degp_hbm.at[c, pl.ds(s * SEG, SEG)])


def _deg_call(src, ew):
    k = functools.partial(
        pl.kernel,
        out_type=jax.ShapeDtypeStruct((NC, NP), jnp.float32),
        mesh=_mesh(),
        compiler_params=pltpu.CompilerParams(needs_layout_passes=False),
        scratch_types=[
            pltpu.VMEM((NP,), jnp.float32),
            pltpu.VMEM((CH_DEG,), jnp.int32),
            pltpu.VMEM((CH_DEG,), jnp.float32),
            pltpu.VMEM_SHARED((NS, NP), jnp.float32),
            pltpu.VMEM((SEG,), jnp.float32),
            pltpu.VMEM((SEG,), jnp.float32),
        ],
    )(_deg_body)
    return k(src, ew)


def _dinv_body(degp_ref, out_ref):
    d = degp_ref[0] + degp_ref[1]
    out_ref[...] = jnp.where(d > 0, lax.rsqrt(d), 0.0)


def _dinv_call(degp):
    out = pl.pallas_call(
        _dinv_body,
        out_shape=jax.ShapeDtypeStruct((NP // 128, 128), jnp.float32),
    )(degp.reshape(NC, NP // 128, 128))
    return out.reshape(NP)


def _stage(src_hbm, dst_hbm, ew_hbm, sidx, didx, wbuf, dinv_buf, off):
    pltpu.sync_copy(src_hbm.at[pl.ds(off, CK)], sidx)
    pltpu.sync_copy(dst_hbm.at[pl.ds(off, CK)], didx)
    pltpu.sync_copy(ew_hbm.at[pl.ds(off, CK)], wbuf)

    def wscale(j, _):
        sl = pl.ds(j * L, L)
        d16 = plsc.load_gather(dinv_buf, [sidx[sl]])
        wbuf[sl] = wbuf[sl] * d16
        return 0
    lax.fori_loop(0, CK // L, wscale, 0, unroll=4)


def _scale_scatter(acc, rows, didx, wbuf):
    def rscale(e, _):
        w16 = plsc.load_gather(wbuf, [jnp.full((L,), e, jnp.int32)])
        for kk in range(D // L):
            sl = pl.ds(kk * L, L)
            rows[e, sl] = rows[e, sl] * w16
        return 0
    lax.fori_loop(0, CK, rscale, 0, unroll=2)
    pltpu.sync_copy(rows, acc.at[didx], add=True)


def _run_core(in_hbm, out_hbm, src_hbm, dst_hbm, ew_hbm, acc,
              rows_a, rows_b, sidx_a, didx_a, wbuf_a, sidx_b, didx_b, wbuf_b,
              dinv_buf, sem_a, sem_b, s):
    zero16 = jnp.zeros((L,), jnp.float32)

    def zrow(r, _):
        for kk in range(D // L):
            rows_a[r, pl.ds(kk * L, L)] = zero16
        return 0
    lax.fori_loop(0, CK, zrow, 0, unroll=2)
    for j in range(ROWS_PER_TILE // RB):
        pltpu.sync_copy(rows_a.at[pl.ds(0, RB)],
                        acc.at[pl.ds(s * ROWS_PER_TILE + j * RB, RB)])
    plsc.subcore_barrier()

    ebase = s * ED_PROP

    _stage(src_hbm, dst_hbm, ew_hbm, sidx_a, didx_a, wbuf_a, dinv_buf, ebase)
    pltpu.async_copy(in_hbm.at[sidx_a], rows_a, sem_a)

    def pair(i2, _):
        off_b = ebase + (2 * i2 + 1) * CK
        _stage(src_hbm, dst_hbm, ew_hbm, sidx_b, didx_b, wbuf_b, dinv_buf,
               off_b)
        pltpu.async_copy(in_hbm.at[sidx_b], rows_b, sem_b)
        pltpu.make_async_copy(in_hbm.at[sidx_a], rows_a, sem_a).wait()
        _scale_scatter(acc, rows_a, didx_a, wbuf_a)

        @pl.when(i2 < PAIRS - 1)
        def _():
            off_a = ebase + (2 * i2 + 2) * CK
            _stage(src_hbm, dst_hbm, ew_hbm, sidx_a, didx_a, wbuf_a, dinv_buf,
                   off_a)
            pltpu.async_copy(in_hbm.at[sidx_a], rows_a, sem_a)

        pltpu.make_async_copy(in_hbm.at[sidx_b], rows_b, sem_b).wait()
        _scale_scatter(acc, rows_b, didx_b, wbuf_b)
        return 0
    lax.fori_loop(0, PAIRS, pair, 0)

    plsc.subcore_barrier()

    for j in range(ROWS_PER_TILE // RB):
        rbase = s * ROWS_PER_TILE + j * RB
        pltpu.sync_copy(acc.at[pl.ds(rbase, RB)], rows_a.at[pl.ds(0, RB)])

        def scrow(r, _):
            g16 = jnp.full((L,), rbase + r, jnp.int32)
            d16 = -plsc.load_gather(dinv_buf, [g16])
            for kk in range(D // L):
                sl = pl.ds(kk * L, L)
                rows_a[r, sl] = rows_a[r, sl] * d16
            return 0
        lax.fori_loop(0, RB, scrow, 0, unroll=2)
        pltpu.sync_copy(rows_a.at[pl.ds(0, RB)], out_hbm.at[pl.ds(rbase, RB)])


def _prop_body(x_hbm, h_hbm, src_hbm, dst_hbm, ew_hbm, dinv_hbm,
               txx_hbm, txh_hbm, acc, rows_a, rows_b, sidx_a, didx_a, wbuf_a,
               sidx_b, didx_b, wbuf_b, dinv_buf, sem_a, sem_b):
    c = lax.axis_index("c")
    s = lax.axis_index("s")
    pltpu.sync_copy(dinv_hbm, dinv_buf)

    @pl.when(c == 0)
    def _():
        _run_core(x_hbm, txx_hbm, src_hbm, dst_hbm, ew_hbm, acc, rows_a,
                  rows_b, sidx_a, didx_a, wbuf_a, sidx_b, didx_b, wbuf_b,
                  dinv_buf, sem_a, sem_b, s)

    @pl.when(c == 1)
    def _():
        _run_core(h_hbm, txh_hbm, src_hbm, dst_hbm, ew_hbm, acc, rows_a,
                  rows_b, sidx_a, didx_a, wbuf_a, sidx_b, didx_b, wbuf_b,
                  dinv_buf, sem_a, sem_b, s)


def _prop_call(x, h, src, dst, ew, dinv):
    k = functools.partial(
        pl.kernel,
        out_type=(jax.ShapeDtypeStruct((NP, D), jnp.float32),
                  jax.ShapeDtypeStruct((NP, D), jnp.float32)),
        mesh=_mesh(),
        compiler_params=pltpu.CompilerParams(needs_layout_passes=False),
        scratch_types=[
            pltpu.VMEM_SHARED((NP, D), jnp.float32),
            pltpu.VMEM((CK, D), jnp.float32),
            pltpu.VMEM((CK, D), jnp.float32),
            pltpu.VMEM((CK,), jnp.int32),
            pltpu.VMEM((CK,), jnp.int32),
            pltpu.VMEM((CK,), jnp.float32),
            pltpu.VMEM((CK,), jnp.int32),
            pltpu.VMEM((CK,), jnp.int32),
            pltpu.VMEM((CK,), jnp.float32),
            pltpu.VMEM((NP,), jnp.float32),
            pltpu.SemaphoreType.DMA,
            pltpu.SemaphoreType.DMA,
        ],
    )(_prop_body)
    return k(x, h, src, dst, ew, dinv)


GB = 2000


def _cell_body(x_ref, tx_ref, h_ref, th_ref, c_ref, wx0, wx1, wh0, wh1, b_ref,
               h_out, c_out):
    pre = (jnp.dot(x_ref[...], wx0[...], preferred_element_type=jnp.float32)
           + jnp.dot(tx_ref[...], wx1[...], preferred_element_type=jnp.float32)
           + jnp.dot(h_ref[...], wh0[...], preferred_element_type=jnp.float32)
           + jnp.dot(th_ref[...], wh1[...], preferred_element_type=jnp.float32)
           + b_ref[...])
    i = jax.nn.sigmoid(pre[:, 0:HID])
    f = jax.nn.sigmoid(pre[:, HID:2 * HID])
    g = jnp.tanh(pre[:, 2 * HID:3 * HID])
    o = jax.nn.sigmoid(pre[:, 3 * HID:4 * HID])
    ct = f * c_ref[...] + i * g
    h_out[...] = o * jnp.tanh(ct)
    c_out[...] = ct


def _cell_call(x, txx, h, txh, c, wx0, wx1, wh0, wh1, b2d):
    row_spec = pl.BlockSpec((GB, D), lambda i: (i, 0))
    w_spec = pl.BlockSpec((D, 4 * HID), lambda i: (0, 0))
    return pl.pallas_call(
        _cell_body,
        grid=(N // GB,),
        in_specs=[row_spec, row_spec, row_spec, row_spec, row_spec,
                  w_spec, w_spec, w_spec, w_spec,
                  pl.BlockSpec((1, 4 * HID), lambda i: (0, 0))],
        out_specs=[pl.BlockSpec((GB, HID), lambda i: (i, 0))] * 2,
        out_shape=[jax.ShapeDtypeStruct((N, HID), jnp.float32)] * 2,
    )(x, txx, h, txh, c, wx0, wx1, wh0, wh1, b2d)


def kernel(x, edge_index, edge_weight, h_prev, c_prev,
           Wxi0, Wxi1, bxi, Whi0, Whi1, bhi,
           Wxf0, Wxf1, bxf, Whf0, Whf1, bhf,
           Wxg0, Wxg1, bxg, Whg0, Whg1, bhg,
           Wxo0, Wxo1, bxo, Who0, Who1, bho):
    pad = E2 - E
    src = jnp.concatenate([edge_index[0], jnp.zeros((pad,), jnp.int32)])
    dst = jnp.concatenate([edge_index[1], jnp.zeros((pad,), jnp.int32)])
    ew = jnp.concatenate([edge_weight, jnp.zeros((pad,), jnp.float32)])

    degp = _deg_call(src, ew)
    dinv = _dinv_call(degp)
    txx, txh = _prop_call(x, h_prev, src, dst, ew, dinv)
    txx = txx[:N]
    txh = txh[:N]

    wx0 = jnp.concatenate([Wxi0, Wxf0, Wxg0, Wxo0], axis=1)
    wx1 = jnp.concatenate([Wxi1, Wxf1, Wxg1, Wxo1], axis=1)
    wh0 = jnp.concatenate([Whi0, Whf0, Whg0, Who0], axis=1)
    wh1 = jnp.concatenate([Whi1, Whf1, Whg1, Who1], axis=1)
    b2d = jnp.concatenate([bxi + bhi, bxf + bhf, bxg + bhg, bxo + bho])[None, :]

    h_t, c_t = _cell_call(x, txx, h_prev, txh, c_prev, wx0, wx1, wh0, wh1, b2d)
    return (h_t, c_t)


# speedup vs baseline: 1.1043x; 1.1043x over previous
import functools

import jax
import jax.numpy as jnp
from jax import lax
from jax.experimental import pallas as pl
from jax.experimental.pallas import tpu as pltpu
from jax.experimental.pallas import tpu_sc as plsc

NC, NS, L = 2, 16, 16
N = 10000
NP = 10240
E = 320000
E2 = 327680
D = 128
HID = 128
ROWS_PER_TILE = NP // NS
RB = 128
ED_DEG = E2 // (NC * NS)
CH_DEG = 2048
CK = 128
NFULL = E2 // NS // CK
NBLK = NFULL // 16
ER = E2 // CK


def _mesh():
    return plsc.VectorSubcoreMesh(core_axis_name="c", subcore_axis_name="s",
                                  num_cores=NC, num_subcores=NS)


def _deg_body(src_hbm, ew_hbm, degp_hbm, acc, idx_buf, w_buf):
    c = lax.axis_index("c")
    s = lax.axis_index("s")
    wid = c * NS + s
    zero16 = jnp.zeros((L,), jnp.float32)

    def zacc(i, _):
        acc[pl.ds(i * L, L)] = zero16
        return 0
    lax.fori_loop(0, NP // L, zacc, 0, unroll=4)

    base = wid * ED_DEG

    def chunk(k, _):
        off = base + k * CH_DEG
        pltpu.sync_copy(src_hbm.at[pl.ds(off, CH_DEG)], idx_buf)
        pltpu.sync_copy(ew_hbm.at[pl.ds(off, CH_DEG)], w_buf)

        def inner(j, _):
            sl = pl.ds(j * L, L)
            plsc.addupdate_scatter(acc, [idx_buf[sl]], w_buf[sl])
            return 0
        lax.fori_loop(0, CH_DEG // L, inner, 0, unroll=4)
        return 0
    lax.fori_loop(0, ED_DEG // CH_DEG, chunk, 0)

    pltpu.sync_copy(acc, degp_hbm.at[wid])


def _deg_call(src, ew):
    k = functools.partial(
        pl.kernel,
        out_type=jax.ShapeDtypeStruct((NC * NS, NP), jnp.float32),
        mesh=_mesh(),
        compiler_params=pltpu.CompilerParams(needs_layout_passes=False),
        scratch_types=[
            pltpu.VMEM((NP,), jnp.float32),
            pltpu.VMEM((CH_DEG,), jnp.int32),
            pltpu.VMEM((CH_DEG,), jnp.float32),
        ],
    )(_deg_body)
    return k(src, ew)


def _dinv_body(degp_ref, out_ref):
    d = jnp.sum(degp_ref[...], axis=0)
    out_ref[...] = jnp.where(d > 0, lax.rsqrt(d), 0.0)


def _dinv_call(degp):
    out = pl.pallas_call(
        _dinv_body,
        out_shape=jax.ShapeDtypeStruct((NP // 128, 128), jnp.float32),
    )(degp.reshape(NC * NS, NP // 128, 128))
    return out.reshape(NP)


CR = CH_DEG // CK


def _wprep_body(src_hbm, ew_hbm, dinv_hbm, w_hbm, dinv_buf, sbuf, ebuf, wbuf):
    c = lax.axis_index("c")
    s = lax.axis_index("s")
    wid = c * NS + s
    pltpu.sync_copy(dinv_hbm, dinv_buf)
    row0 = wid * (ED_DEG // CK)

    def chunk(k, _):
        off = pl.ds(row0 + k * CR, CR)
        pltpu.sync_copy(src_hbm.at[off], sbuf)
        pltpu.sync_copy(ew_hbm.at[off], ebuf)

        def inner(j, _):
            r = j // (CK // L)
            i = j % (CK // L)
            sl = pl.ds(i * L, L)
            d16 = plsc.load_gather(dinv_buf, [sbuf[r, sl]])
            wbuf[r, sl] = -ebuf[r, sl] * d16
            return 0
        lax.fori_loop(0, CH_DEG // L, inner, 0, unroll=4)
        pltpu.sync_copy(wbuf, w_hbm.at[off])
        return 0
    lax.fori_loop(0, ED_DEG // CH_DEG, chunk, 0)


def _wprep_call(src2d, ew2d, dinv):
    k = functools.partial(
        pl.kernel,
        out_type=jax.ShapeDtypeStruct((ER, CK), jnp.float32),
        mesh=_mesh(),
        compiler_params=pltpu.CompilerParams(needs_layout_passes=False),
        scratch_types=[
            pltpu.VMEM((NP,), jnp.float32),
            pltpu.VMEM((CR, CK), jnp.int32),
            pltpu.VMEM((CR, CK), jnp.float32),
            pltpu.VMEM((CR, CK), jnp.float32),
        ],
    )(_wprep_body)
    return k(src2d, ew2d, dinv)


def _gather(in_hbm, sbuf, jj, rows, sem):
    pltpu.async_copy(in_hbm.at[sbuf.at[jj]], rows, sem)


def _gwait(in_hbm, sbuf, jj, rows, sem):
    pltpu.make_async_copy(in_hbm.at[sbuf.at[jj]], rows, sem).wait()


def _rscale_scatter(acc, rows, wsrc, dbuf, jj):
    jj16 = jnp.full((L,), jj, jnp.int32)

    def rs(e, _):
        w16 = plsc.load_gather(wsrc, [jj16, jnp.full((L,), e, jnp.int32)])
        for kk in range(D // L):
            sl = pl.ds(kk * L, L)
            rows[e, sl] = rows[e, sl] * w16
        return 0
    lax.fori_loop(0, CK, rs, 0, unroll=2)
    pltpu.sync_copy(rows, acc.at[dbuf.at[jj]], add=True)


def _run_core(in_hbm, out_hbm, src_hbm, dst_hbm, w_hbm, dinv_hbm, acc,
              rows_a, rows_b, sbuf_a, dbuf_a, wsrc_a, sbuf_b, dbuf_b, wsrc_b,
              dslice, sem_ga, sem_gb, sem_ea, sem_eb, s):
    zero16 = jnp.zeros((L,), jnp.float32)
    pltpu.sync_copy(dinv_hbm.at[pl.ds(s * ROWS_PER_TILE, ROWS_PER_TILE)],
                    dslice)

    def zrow(r, _):
        for kk in range(D // L):
            rows_a[r, pl.ds(kk * L, L)] = zero16
        return 0
    lax.fori_loop(0, CK, zrow, 0, unroll=2)
    for j in range(ROWS_PER_TILE // RB):
        pltpu.sync_copy(rows_a.at[pl.ds(0, RB)],
                        acc.at[pl.ds(s * ROWS_PER_TILE + j * RB, RB)])
    plsc.subcore_barrier()

    brow = s * NFULL

    def stage(b, sbuf, dbuf, wsrc, sem):
        off = pl.ds(brow + b * 16, 16)
        pltpu.async_copy(src_hbm.at[off], sbuf, sem)
        pltpu.async_copy(dst_hbm.at[off], dbuf, sem)
        pltpu.async_copy(w_hbm.at[off], wsrc, sem)

    def stage_wait(b, sbuf, dbuf, wsrc, sem):
        off = pl.ds(brow + b * 16, 16)
        pltpu.make_async_copy(src_hbm.at[off], sbuf, sem).wait()
        pltpu.make_async_copy(dst_hbm.at[off], dbuf, sem).wait()
        pltpu.make_async_copy(w_hbm.at[off], wsrc, sem).wait()

    stage(0, sbuf_a, dbuf_a, wsrc_a, sem_ea)
    for b in range(NBLK):
        if b % 2 == 0:
            sbuf, dbuf, wsrc, sem_e = sbuf_a, dbuf_a, wsrc_a, sem_ea
        else:
            sbuf, dbuf, wsrc, sem_e = sbuf_b, dbuf_b, wsrc_b, sem_eb
        if b < NBLK - 1:
            if b % 2 == 0:
                stage(b + 1, sbuf_b, dbuf_b, wsrc_b, sem_eb)
            else:
                stage(b + 1, sbuf_a, dbuf_a, wsrc_a, sem_ea)
        stage_wait(b, sbuf, dbuf, wsrc, sem_e)

        _gather(in_hbm, sbuf, 0, rows_a, sem_ga)

        def pair(i, _):
            jj = 2 * i
            _gather(in_hbm, sbuf, jj + 1, rows_b, sem_gb)
            _gwait(in_hbm, sbuf, jj, rows_a, sem_ga)
            _rscale_scatter(acc, rows_a, wsrc, dbuf, jj)

            @pl.when(i < 7)
            def _():
                _gather(in_hbm, sbuf, jj + 2, rows_a, sem_ga)

            _gwait(in_hbm, sbuf, jj + 1, rows_b, sem_gb)
            _rscale_scatter(acc, rows_b, wsrc, dbuf, jj + 1)
            return 0
        lax.fori_loop(0, 8, pair, 0)

    plsc.subcore_barrier()

    for j in range(ROWS_PER_TILE // RB):
        rbase = s * ROWS_PER_TILE + j * RB
        pltpu.sync_copy(acc.at[pl.ds(rbase, RB)], rows_a.at[pl.ds(0, RB)])

        def scrow(r, _):
            g16 = jnp.full((L,), j * RB + r, jnp.int32)
            d16 = plsc.load_gather(dslice, [g16])
            for kk in range(D // L):
                sl = pl.ds(kk * L, L)
                rows_a[r, sl] = rows_a[r, sl] * d16
            return 0
        lax.fori_loop(0, RB, scrow, 0, unroll=2)
        pltpu.sync_copy(rows_a.at[pl.ds(0, RB)], out_hbm.at[pl.ds(rbase, RB)])


def _prop_body(x_hbm, h_hbm, src_hbm, dst_hbm, w_hbm, dinv_hbm,
               txx_hbm, txh_hbm, acc, rows_a, rows_b,
               sbuf_a, dbuf_a, wsrc_a, sbuf_b, dbuf_b, wsrc_b,
               dslice, sem_ga, sem_gb, sem_ea, sem_eb):
    c = lax.axis_index("c")
    s = lax.axis_index("s")

    @pl.when(c == 0)
    def _():
        _run_core(x_hbm, txx_hbm, src_hbm, dst_hbm, w_hbm, dinv_hbm, acc,
                  rows_a, rows_b, sbuf_a, dbuf_a, wsrc_a, sbuf_b, dbuf_b,
                  wsrc_b, dslice, sem_ga, sem_gb, sem_ea, sem_eb, s)

    @pl.when(c == 1)
    def _():
        _run_core(h_hbm, txh_hbm, src_hbm, dst_hbm, w_hbm, dinv_hbm, acc,
                  rows_a, rows_b, sbuf_a, dbuf_a, wsrc_a, sbuf_b, dbuf_b,
                  wsrc_b, dslice, sem_ga, sem_gb, sem_ea, sem_eb, s)


def _prop_call(x, h, src2d, dst2d, w2d, dinv):
    k = functools.partial(
        pl.kernel,
        out_type=(jax.ShapeDtypeStruct((NP, D), jnp.float32),
                  jax.ShapeDtypeStruct((NP, D), jnp.float32)),
        mesh=_mesh(),
        compiler_params=pltpu.CompilerParams(needs_layout_passes=False),
        scratch_types=[
            pltpu.VMEM_SHARED((NP, D), jnp.float32),
            pltpu.VMEM((CK, D), jnp.float32),
            pltpu.VMEM((CK, D), jnp.float32),
            pltpu.VMEM((16, CK), jnp.int32),
            pltpu.VMEM((16, CK), jnp.int32),
            pltpu.VMEM((16, CK), jnp.float32),
            pltpu.VMEM((16, CK), jnp.int32),
            pltpu.VMEM((16, CK), jnp.int32),
            pltpu.VMEM((16, CK), jnp.float32),
            pltpu.VMEM((ROWS_PER_TILE,), jnp.float32),
            pltpu.SemaphoreType.DMA,
            pltpu.SemaphoreType.DMA,
            pltpu.SemaphoreType.DMA,
            pltpu.SemaphoreType.DMA,
        ],
    )(_prop_body)
    return k(x, h, src2d, dst2d, w2d, dinv)


GB = 2000


def _cell_body(x_ref, tx_ref, h_ref, th_ref, c_ref, wx0, wx1, wh0, wh1, b_ref,
               h_out, c_out):
    pre = (jnp.dot(x_ref[...], wx0[...], preferred_element_type=jnp.float32)
           + jnp.dot(tx_ref[...], wx1[...], preferred_element_type=jnp.float32)
           + jnp.dot(h_ref[...], wh0[...], preferred_element_type=jnp.float32)
           + jnp.dot(th_ref[...], wh1[...], preferred_element_type=jnp.float32)
           + b_ref[...])
    i = jax.nn.sigmoid(pre[:, 0:HID])
    f = jax.nn.sigmoid(pre[:, HID:2 * HID])
    g = jnp.tanh(pre[:, 2 * HID:3 * HID])
    o = jax.nn.sigmoid(pre[:, 3 * HID:4 * HID])
    ct = f * c_ref[...] + i * g
    h_out[...] = o * jnp.tanh(ct)
    c_out[...] = ct


def _cell_call(x, txx, h, txh, c, wx0, wx1, wh0, wh1, b2d):
    row_spec = pl.BlockSpec((GB, D), lambda i: (i, 0))
    w_spec = pl.BlockSpec((D, 4 * HID), lambda i: (0, 0))
    return pl.pallas_call(
        _cell_body,
        grid=(N // GB,),
        in_specs=[row_spec, row_spec, row_spec, row_spec, row_spec,
                  w_spec, w_spec, w_spec, w_spec,
                  pl.BlockSpec((1, 4 * HID), lambda i: (0, 0))],
        out_specs=[pl.BlockSpec((GB, HID), lambda i: (i, 0))] * 2,
        out_shape=[jax.ShapeDtypeStruct((N, HID), jnp.float32)] * 2,
    )(x, txx, h, txh, c, wx0, wx1, wh0, wh1, b2d)


def kernel(x, edge_index, edge_weight, h_prev, c_prev,
           Wxi0, Wxi1, bxi, Whi0, Whi1, bhi,
           Wxf0, Wxf1, bxf, Whf0, Whf1, bhf,
           Wxg0, Wxg1, bxg, Whg0, Whg1, bhg,
           Wxo0, Wxo1, bxo, Who0, Who1, bho):
    pad = E2 - E
    src = jnp.concatenate([edge_index[0], jnp.zeros((pad,), jnp.int32)])
    dst = jnp.concatenate([edge_index[1], jnp.zeros((pad,), jnp.int32)])
    ew = jnp.concatenate([edge_weight, jnp.zeros((pad,), jnp.float32)])
    src2d = src.reshape(ER, CK)
    dst2d = dst.reshape(ER, CK)
    ew2d = ew.reshape(ER, CK)

    degp = _deg_call(src, ew)
    dinv = _dinv_call(degp)
    w2d = _wprep_call(src2d, ew2d, dinv)
    txx, txh = _prop_call(x, h_prev, src2d, dst2d, w2d, dinv)
    txx = txx[:N]
    txh = txh[:N]

    wx0 = jnp.concatenate([Wxi0, Wxf0, Wxg0, Wxo0], axis=1)
    wx1 = jnp.concatenate([Wxi1, Wxf1, Wxg1, Wxo1], axis=1)
    wh0 = jnp.concatenate([Whi0, Whf0, Whg0, Who0], axis=1)
    wh1 = jnp.concatenate([Whi1, Whf1, Whg1, Who1], axis=1)
    b2d = jnp.concatenate([bxi + bhi, bxf + bhf, bxg + bhg, bxo + bho])[None, :]

    h_t, c_t = _cell_call(x, txx, h_prev, txh, c_prev, wx0, wx1, wh0, wh1, b2d)
    return (h_t, c_t)
